# Initial kernel scaffold; baseline (speedup 1.0000x reference)
import jax
import jax.numpy as jnp
from jax.experimental import pallas as pl


def kernel(x, w1, g1, b1, w2, g2, b2, w3, g3, b3, ws, gs, bs):
    raise NotImplementedError("write your pallas kernel here")


# speedup vs baseline: 4.3192x; 4.3192x over previous
import functools

import jax
import jax.numpy as jnp
from jax import lax
from jax.experimental import pallas as pl
from jax.experimental.pallas import tpu as pltpu

_EPS = 1e-5
_VMEM_LIMIT = 64 * 1024 * 1024


def _cparams():
    return pltpu.CompilerParams(
        dimension_semantics=("parallel",),
        vmem_limit_bytes=_VMEM_LIMIT)


def _affine(s_parts, q_parts, gamma, beta, count):
    s = jnp.sum(s_parts, axis=(0, 1))
    q = jnp.sum(q_parts, axis=(0, 1))
    mean = s / count
    var = jnp.maximum(q / count - mean * mean, 0.0)
    scale = gamma.reshape(-1) * lax.rsqrt(var + _EPS)
    shift = beta.reshape(-1) - mean * scale
    return scale.reshape(1, -1), shift.reshape(1, -1)


def _pass_a_kernel(x_ref, w1_ref, ws_ref,
                   h_ref, s1_ref, q1_ref, sr_ref, qr_ref):
    xb = x_ref[...]
    h = jnp.dot(xb, w1_ref[...], preferred_element_type=jnp.float32)
    r = jnp.dot(xb, ws_ref[...], preferred_element_type=jnp.float32)
    h_ref[...] = h
    s1_ref[...] = jnp.sum(h, axis=0, keepdims=True)[None]
    q1_ref[...] = jnp.sum(h * h, axis=0, keepdims=True)[None]
    sr_ref[...] = jnp.sum(r, axis=0, keepdims=True)[None]
    qr_ref[...] = jnp.sum(r * r, axis=0, keepdims=True)[None]


def _pass_a(x2d, w1, ws, tm):
    m, c0 = x2d.shape
    cm = w1.shape[1]
    ce = ws.shape[1]
    nt = m // tm
    return pl.pallas_call(
        _pass_a_kernel,
        grid=(nt,),
        in_specs=[pl.BlockSpec((tm, c0), lambda i: (i, 0)),
                  pl.BlockSpec((c0, cm), lambda i: (0, 0)),
                  pl.BlockSpec((c0, ce), lambda i: (0, 0))],
        out_specs=(pl.BlockSpec((tm, cm), lambda i: (i, 0)),
                   pl.BlockSpec((1, 1, cm), lambda i: (i, 0, 0)),
                   pl.BlockSpec((1, 1, cm), lambda i: (i, 0, 0)),
                   pl.BlockSpec((1, 1, ce), lambda i: (i, 0, 0)),
                   pl.BlockSpec((1, 1, ce), lambda i: (i, 0, 0))),
        out_shape=(jax.ShapeDtypeStruct((m, cm), jnp.float32),
                   jax.ShapeDtypeStruct((nt, 1, cm), jnp.float32),
                   jax.ShapeDtypeStruct((nt, 1, cm), jnp.float32),
                   jax.ShapeDtypeStruct((nt, 1, ce), jnp.float32),
                   jax.ShapeDtypeStruct((nt, 1, ce), jnp.float32)),
        compiler_params=_cparams(),
        cost_estimate=pl.CostEstimate(
            flops=2 * m * c0 * (cm + ce), transcendentals=0,
            bytes_accessed=4 * (m * c0 + m * cm)),
    )(x2d, w1, ws)


def _pass_b_kernel(x_ref, w_ref, sc_ref, sh_ref,
                   y_ref, s_ref, q_ref, pad_ref, col_ref):
    _, h, w, c = x_ref.shape
    cout = w_ref.shape[1]
    a = jnp.maximum(x_ref[0] * sc_ref[0] + sh_ref[0], 0.0)

    pad_ref[1:h + 1, 1:w + 1, :] = a
    zrow = jnp.zeros((1, w + 2, c), jnp.float32)
    pad_ref[0:1, :, :] = zrow
    pad_ref[h + 1:h + 2, :, :] = zrow
    zcol = jnp.zeros((h, 1, c), jnp.float32)
    pad_ref[1:h + 1, 0:1, :] = zcol
    pad_ref[1:h + 1, w + 1:w + 2, :] = zcol

    for kh in range(3):
        for kw in range(3):
            t = kh * 3 + kw
            col_ref[:, t * c:(t + 1) * c] = (
                pad_ref[kh:kh + h, kw:kw + w, :].reshape(h * w, c))

    y = jnp.dot(col_ref[...], w_ref[...], preferred_element_type=jnp.float32)
    y_ref[...] = y.reshape(1, h, w, cout)
    s_ref[...] = jnp.sum(y, axis=0, keepdims=True)[None]
    q_ref[...] = jnp.sum(y * y, axis=0, keepdims=True)[None]


def _pass_b(x4d, w2f, aff):
    n, h, w, c = x4d.shape
    cout = w2f.shape[1]
    sc, sh = aff
    return pl.pallas_call(
        _pass_b_kernel,
        grid=(n,),
        in_specs=[pl.BlockSpec((1, h, w, c), lambda i: (i, 0, 0, 0)),
                  pl.BlockSpec((9 * c, cout), lambda i: (0, 0)),
                  pl.BlockSpec((1, c), lambda i: (0, 0)),
                  pl.BlockSpec((1, c), lambda i: (0, 0))],
        out_specs=(pl.BlockSpec((1, h, w, cout), lambda i: (i, 0, 0, 0)),
                   pl.BlockSpec((1, 1, cout), lambda i: (i, 0, 0)),
                   pl.BlockSpec((1, 1, cout), lambda i: (i, 0, 0))),
        out_shape=(jax.ShapeDtypeStruct((n, h, w, cout), jnp.float32),
                   jax.ShapeDtypeStruct((n, 1, cout), jnp.float32),
                   jax.ShapeDtypeStruct((n, 1, cout), jnp.float32)),
        scratch_shapes=[pltpu.VMEM((h + 2, w + 2, c), jnp.float32),
                        pltpu.VMEM((h * w, 9 * c), jnp.float32)],
        compiler_params=_cparams(),
        cost_estimate=pl.CostEstimate(
            flops=2 * n * h * w * 9 * c * cout, transcendentals=0,
            bytes_accessed=4 * (n * h * w * c + 9 * c * cout
                                + n * h * w * cout)),
    )(x4d, w2f, sc, sh)


def _pass_c_kernel(h2_ref, w3_ref, sc_ref, sh_ref, s_ref, q_ref):
    t = jnp.maximum(h2_ref[...] * sc_ref[...] + sh_ref[...], 0.0)
    z = jnp.dot(t, w3_ref[...], preferred_element_type=jnp.float32)
    s_ref[...] = jnp.sum(z, axis=0, keepdims=True)[None]
    q_ref[...] = jnp.sum(z * z, axis=0, keepdims=True)[None]


def _pass_c(h2d, w3, aff, tm):
    m, cm = h2d.shape
    ce = w3.shape[1]
    nt = m // tm
    sc, sh = aff
    return pl.pallas_call(
        _pass_c_kernel,
        grid=(nt,),
        in_specs=[pl.BlockSpec((tm, cm), lambda i: (i, 0)),
                  pl.BlockSpec((cm, ce), lambda i: (0, 0)),
                  pl.BlockSpec((1, cm), lambda i: (0, 0)),
                  pl.BlockSpec((1, cm), lambda i: (0, 0))],
        out_specs=(pl.BlockSpec((1, 1, ce), lambda i: (i, 0, 0)),
                   pl.BlockSpec((1, 1, ce), lambda i: (i, 0, 0))),
        out_shape=(jax.ShapeDtypeStruct((nt, 1, ce), jnp.float32),
                   jax.ShapeDtypeStruct((nt, 1, ce), jnp.float32)),
        compiler_params=_cparams(),
        cost_estimate=pl.CostEstimate(
            flops=2 * m * cm * ce, transcendentals=0,
            bytes_accessed=4 * (m * cm + cm * ce)),
    )(h2d, w3, sc, sh)


def _pass_d_kernel(h2_ref, x_ref, w3_ref, ws_ref,
                   sc2_ref, sh2_ref, sc3_ref, sh3_ref, scs_ref, shs_ref,
                   o_ref):
    t = jnp.maximum(h2_ref[...] * sc2_ref[...] + sh2_ref[...], 0.0)
    z = jnp.dot(t, w3_ref[...], preferred_element_type=jnp.float32)
    r = jnp.dot(x_ref[...], ws_ref[...], preferred_element_type=jnp.float32)
    o = (z * sc3_ref[...] + sh3_ref[...]) + (r * scs_ref[...] + shs_ref[...])
    o_ref[...] = jnp.maximum(o, 0.0).astype(o_ref.dtype)


def _pass_d(h2d, x2d, w3, ws, aff2, aff3, affs, out_dtype, tm):
    m, cm = h2d.shape
    c0 = x2d.shape[1]
    ce = w3.shape[1]
    nt = m // tm
    vec = lambda a: pl.BlockSpec((1, a.shape[1]), lambda i: (0, 0))
    args = [h2d, x2d, w3, ws, aff2[0], aff2[1], aff3[0], aff3[1],
            affs[0], affs[1]]
    return pl.pallas_call(
        _pass_d_kernel,
        grid=(nt,),
        in_specs=[pl.BlockSpec((tm, cm), lambda i: (i, 0)),
                  pl.BlockSpec((tm, c0), lambda i: (i, 0)),
                  pl.BlockSpec((cm, ce), lambda i: (0, 0)),
                  pl.BlockSpec((c0, ce), lambda i: (0, 0)),
                  vec(aff2[0]), vec(aff2[1]), vec(aff3[0]), vec(aff3[1]),
                  vec(affs[0]), vec(affs[1])],
        out_specs=pl.BlockSpec((tm, ce), lambda i: (i, 0)),
        out_shape=jax.ShapeDtypeStruct((m, ce), out_dtype),
        compiler_params=_cparams(),
        cost_estimate=pl.CostEstimate(
            flops=2 * m * (cm + c0) * ce, transcendentals=0,
            bytes_accessed=4 * (m * cm + m * c0 + m * ce)),
    )(*args)


def kernel(x, w1, g1, b1, w2, g2, b2, w3, g3, b3, ws, gs, bs):
    n, h, w, c0 = x.shape
    cm = w1.shape[1]
    ce = w3.shape[1]
    m = n * h * w
    tm = 4096 if m % 4096 == 0 else m

    x2d = x.reshape(m, c0)
    w2f = w2.reshape(9 * cm, cm)

    h1, s1, q1, sr, qr = _pass_a(x2d, w1, ws, tm)
    aff1 = _affine(s1, q1, g1, b1, m)
    affs = _affine(sr, qr, gs, bs, m)

    h2, s2, q2 = _pass_b(h1.reshape(n, h, w, cm), w2f, aff1)
    aff2 = _affine(s2, q2, g2, b2, m)

    h2d = h2.reshape(m, cm)
    s3, q3 = _pass_c(h2d, w3, aff2, tm)
    aff3 = _affine(s3, q3, g3, b3, m)

    y2d = _pass_d(h2d, x2d, w3, ws, aff2, aff3, affs, x.dtype, tm)
    return y2d.reshape(n, h, w, ce)
